# baseline (speedup 1.0000x reference)
import jax
import jax.numpy as jnp
from jax import lax
from jax.experimental import pallas as pl
from jax.experimental.pallas import tpu as pltpu
from jax.experimental.pallas import tpu_sc as plsc

B, L, H = 4096, 50, 768
LP = 56
HH = H // 2
NC, NS = 2, 16
NW = NC * NS
ROWS_PER_W = B // NW
LANES = 16
HVH = HH // LANES
NUM_TYPES = 1000
REP = 32


def _body(emb_hbm, idsp_hbm, tab0_hbm, tab1_hbm, out_hbm,
          idx_v, buf_e, buf_g, sem_i, sem_e, sem_g, sem_o):
    wid = lax.axis_index("s") * NC + lax.axis_index("c")
    base = wid * ROWS_PER_W
    tabs = (tab0_hbm, tab1_hbm)

    def start_idx(r):
        pltpu.async_copy(idsp_hbm.at[pl.ds((base + r) * LP, LP)],
                         idx_v.at[r % 4], sem_i.at[r % 4])

    def wait_idx(r):
        pltpu.make_async_copy(idsp_hbm.at[pl.ds((base + r) * LP, LP)],
                              idx_v.at[r % 4], sem_i.at[r % 4]).wait()

    def start_gather(r, h):
        pltpu.async_copy(tabs[h].at[idx_v.at[r % 4]], buf_g.at[h],
                         sem_g.at[h])

    def wait_gather(r, h):
        pltpu.make_async_copy(tabs[h].at[idx_v.at[r % 4]], buf_g.at[h],
                              sem_g.at[h]).wait()

    def start_emb(r, b):
        pltpu.async_copy(emb_hbm.at[base + r], buf_e.at[b], sem_e.at[b])

    def wait_emb(r, b):
        pltpu.make_async_copy(emb_hbm.at[base + r], buf_e.at[b],
                              sem_e.at[b]).wait()

    def start_out(r, b):
        pltpu.async_copy(buf_e.at[b], out_hbm.at[base + r], sem_o.at[b])

    def wait_out(r, b):
        pltpu.make_async_copy(buf_e.at[b], out_hbm.at[base + r],
                              sem_o.at[b]).wait()

    start_idx(0)
    start_idx(1)
    wait_idx(0)
    start_emb(0, 0)
    start_gather(0, 0)
    start_gather(0, 1)
    wait_idx(1)

    def outer(r2, carry):
        for b in range(2):
            r = r2 * 2 + b

            @pl.when(r + 2 < ROWS_PER_W)
            def _idx_prefetch():
                start_idx(r + 2)

            @pl.when(r + 1 < ROWS_PER_W)
            def _emb_prefetch():
                @pl.when(r >= 1)
                def _drain():
                    wait_out(r - 1, 1 - b)
                start_emb(r + 1, 1 - b)

            wait_emb(r, b)

            for h in range(2):
                wait_gather(r, h)

                def tok(l, carry2):
                    for j in range(HVH):
                        plsc.addupdate(
                            buf_e.at[b, l, pl.ds(h * HH + j * LANES, LANES)],
                            buf_g[h, l, pl.ds(j * LANES, LANES)])
                    return carry2

                lax.fori_loop(0, L, tok, 0)

                @pl.when(r + 1 < ROWS_PER_W)
                def _gather_next():
                    start_gather(r + 1, h)

            start_out(r, b)

            @pl.when(r + 2 < ROWS_PER_W)
            def _idx_ready():
                wait_idx(r + 2)
        return carry

    lax.fori_loop(0, ROWS_PER_W // 2, outer, 0)
    wait_out(ROWS_PER_W - 2, 0)
    wait_out(ROWS_PER_W - 1, 1)


@jax.jit
def kernel(batch_Phrase_emb, Phrase_type_ids, phrase_attribute_emb_all):
    ids = Phrase_type_ids.astype(jnp.int32)
    ids_pad = jnp.pad(ids, ((0, 0), (0, LP - L)), constant_values=1000)
    worker = (jnp.arange(B, dtype=jnp.int32) // ROWS_PER_W) % REP
    ids_off = (ids_pad + (worker * (NUM_TYPES + 1))[:, None]).reshape(-1)
    tab_rep = jnp.tile(phrase_attribute_emb_all, (REP, 1))
    tab0 = tab_rep[:, :HH]
    tab1 = tab_rep[:, HH:]

    run = pl.kernel(
        _body,
        out_type=jax.ShapeDtypeStruct((B, L, H), jnp.float32),
        mesh=plsc.VectorSubcoreMesh(core_axis_name="c", subcore_axis_name="s"),
        scratch_types=[
            pltpu.VMEM((4, LP), jnp.int32),
            pltpu.VMEM((2, L, H), jnp.float32),
            pltpu.VMEM((2, LP, HH), jnp.float32),
            pltpu.SemaphoreType.DMA((4,)),
            pltpu.SemaphoreType.DMA((2,)),
            pltpu.SemaphoreType.DMA((2,)),
            pltpu.SemaphoreType.DMA((2,)),
        ],
    )
    return run(batch_Phrase_emb, ids_off, tab0, tab1)


# speedup vs baseline: 1.0524x; 1.0524x over previous
import jax
import jax.numpy as jnp
from jax import lax
from jax.experimental import pallas as pl
from jax.experimental.pallas import tpu as pltpu
from jax.experimental.pallas import tpu_sc as plsc

B, L, H = 4096, 50, 768
LP = 56
HH = H // 2
NC, NS = 2, 16
NW = NC * NS
ROWS_PER_W = B // NW
LANES = 16
HVH = HH // LANES
NUM_TYPES = 1000
REP = 16


def _body(emb_hbm, idsp_hbm, tab0_hbm, tab1_hbm, out_hbm,
          idx_v, buf_e, buf_g, sem_i, sem_e, sem_g, sem_o):
    wid = lax.axis_index("s") * NC + lax.axis_index("c")
    base = wid * ROWS_PER_W
    tabs = (tab0_hbm, tab1_hbm)

    def start_idx(r):
        pltpu.async_copy(idsp_hbm.at[pl.ds((base + r) * LP, LP)],
                         idx_v.at[r % 4], sem_i.at[r % 4])

    def wait_idx(r):
        pltpu.make_async_copy(idsp_hbm.at[pl.ds((base + r) * LP, LP)],
                              idx_v.at[r % 4], sem_i.at[r % 4]).wait()

    def start_gather(r, h):
        pltpu.async_copy(tabs[h].at[idx_v.at[r % 4]], buf_g.at[h],
                         sem_g.at[h])

    def wait_gather(r, h):
        pltpu.make_async_copy(tabs[h].at[idx_v.at[r % 4]], buf_g.at[h],
                              sem_g.at[h]).wait()

    def start_emb(r, b):
        pltpu.async_copy(emb_hbm.at[base + r], buf_e.at[b], sem_e.at[b])

    def wait_emb(r, b):
        pltpu.make_async_copy(emb_hbm.at[base + r], buf_e.at[b],
                              sem_e.at[b]).wait()

    def start_out(r, b):
        pltpu.async_copy(buf_e.at[b], out_hbm.at[base + r], sem_o.at[b])

    def wait_out(r, b):
        pltpu.make_async_copy(buf_e.at[b], out_hbm.at[base + r],
                              sem_o.at[b]).wait()

    start_idx(0)
    start_idx(1)
    wait_idx(0)
    start_emb(0, 0)
    start_gather(0, 0)
    start_gather(0, 1)
    wait_idx(1)

    def outer(r2, carry):
        for b in range(2):
            r = r2 * 2 + b

            @pl.when(r + 2 < ROWS_PER_W)
            def _idx_prefetch():
                start_idx(r + 2)

            @pl.when(r + 1 < ROWS_PER_W)
            def _emb_prefetch():
                @pl.when(r >= 1)
                def _drain():
                    wait_out(r - 1, 1 - b)
                start_emb(r + 1, 1 - b)

            wait_emb(r, b)

            for h in range(2):
                wait_gather(r, h)

                def tok(l, carry2):
                    for j in range(HVH):
                        plsc.addupdate(
                            buf_e.at[b, l, pl.ds(h * HH + j * LANES, LANES)],
                            buf_g[h, l, pl.ds(j * LANES, LANES)])
                    return carry2

                lax.fori_loop(0, L, tok, 0)

                @pl.when(r + 1 < ROWS_PER_W)
                def _gather_next():
                    start_gather(r + 1, h)

            start_out(r, b)

            @pl.when(r + 2 < ROWS_PER_W)
            def _idx_ready():
                wait_idx(r + 2)
        return carry

    lax.fori_loop(0, ROWS_PER_W // 2, outer, 0)
    wait_out(ROWS_PER_W - 2, 0)
    wait_out(ROWS_PER_W - 1, 1)


@jax.jit
def kernel(batch_Phrase_emb, Phrase_type_ids, phrase_attribute_emb_all):
    ids = Phrase_type_ids.astype(jnp.int32)
    ids_pad = jnp.pad(ids, ((0, 0), (0, LP - L)), constant_values=1000)
    worker = (jnp.arange(B, dtype=jnp.int32) // ROWS_PER_W) % REP
    ids_off = (ids_pad + (worker * (NUM_TYPES + 1))[:, None]).reshape(-1)
    tab0 = jnp.tile(phrase_attribute_emb_all[:, :HH], (REP, 1))
    tab1 = jnp.tile(phrase_attribute_emb_all[:, HH:], (REP, 1))

    run = pl.kernel(
        _body,
        out_type=jax.ShapeDtypeStruct((B, L, H), jnp.float32),
        mesh=plsc.VectorSubcoreMesh(core_axis_name="c", subcore_axis_name="s"),
        scratch_types=[
            pltpu.VMEM((4, LP), jnp.int32),
            pltpu.VMEM((2, L, H), jnp.float32),
            pltpu.VMEM((2, LP, HH), jnp.float32),
            pltpu.SemaphoreType.DMA((4,)),
            pltpu.SemaphoreType.DMA((2,)),
            pltpu.SemaphoreType.DMA((2,)),
            pltpu.SemaphoreType.DMA((2,)),
        ],
    )
    return run(batch_Phrase_emb, ids_off, tab0, tab1)


# speedup vs baseline: 1.0674x; 1.0143x over previous
import jax
import jax.numpy as jnp
from jax import lax
from jax.experimental import pallas as pl
from jax.experimental.pallas import tpu as pltpu
from jax.experimental.pallas import tpu_sc as plsc

B, L, H = 4096, 50, 768
LP = 56
HH = H // 2
NC, NS = 2, 16
NW = NC * NS
ROWS_PER_W = B // NW
LANES = 16
HVH = HH // LANES
NUM_TYPES = 1000
REP = 8


def _body(emb_hbm, idsp_hbm, tab0_hbm, tab1_hbm, out_hbm,
          idx_v, buf_e, buf_g, sem_i, sem_e, sem_g, sem_o):
    wid = lax.axis_index("s") * NC + lax.axis_index("c")
    base = wid * ROWS_PER_W
    tabs = (tab0_hbm, tab1_hbm)

    def start_idx(r):
        pltpu.async_copy(idsp_hbm.at[pl.ds((base + r) * LP, LP)],
                         idx_v.at[r % 4], sem_i.at[r % 4])

    def wait_idx(r):
        pltpu.make_async_copy(idsp_hbm.at[pl.ds((base + r) * LP, LP)],
                              idx_v.at[r % 4], sem_i.at[r % 4]).wait()

    def start_gather(r, h):
        pltpu.async_copy(tabs[h].at[idx_v.at[r % 4, pl.ds(0, L)]],
                         buf_g.at[h], sem_g.at[h])

    def wait_gather(r, h):
        pltpu.make_async_copy(tabs[h].at[idx_v.at[r % 4, pl.ds(0, L)]],
                              buf_g.at[h], sem_g.at[h]).wait()

    def start_emb(r, b):
        pltpu.async_copy(emb_hbm.at[base + r], buf_e.at[b], sem_e.at[b])

    def wait_emb(r, b):
        pltpu.make_async_copy(emb_hbm.at[base + r], buf_e.at[b],
                              sem_e.at[b]).wait()

    def start_out(r, b):
        pltpu.async_copy(buf_e.at[b], out_hbm.at[base + r], sem_o.at[b])

    def wait_out(r, b):
        pltpu.make_async_copy(buf_e.at[b], out_hbm.at[base + r],
                              sem_o.at[b]).wait()

    start_idx(0)
    start_idx(1)
    wait_idx(0)
    start_emb(0, 0)
    start_gather(0, 0)
    start_gather(0, 1)
    wait_idx(1)

    def outer(r2, carry):
        for b in range(2):
            r = r2 * 2 + b

            @pl.when(r + 2 < ROWS_PER_W)
            def _idx_prefetch():
                start_idx(r + 2)

            @pl.when(r + 1 < ROWS_PER_W)
            def _emb_prefetch():
                @pl.when(r >= 1)
                def _drain():
                    wait_out(r - 1, 1 - b)
                start_emb(r + 1, 1 - b)

            wait_emb(r, b)

            for h in range(2):
                wait_gather(r, h)

                def tok(l, carry2):
                    for j in range(HVH):
                        plsc.addupdate(
                            buf_e.at[b, l, pl.ds(h * HH + j * LANES, LANES)],
                            buf_g[h, l, pl.ds(j * LANES, LANES)])
                    return carry2

                lax.fori_loop(0, L, tok, 0)

                @pl.when(r + 1 < ROWS_PER_W)
                def _gather_next():
                    start_gather(r + 1, h)

            start_out(r, b)

            @pl.when(r + 2 < ROWS_PER_W)
            def _idx_ready():
                wait_idx(r + 2)
        return carry

    lax.fori_loop(0, ROWS_PER_W // 2, outer, 0)
    wait_out(ROWS_PER_W - 2, 0)
    wait_out(ROWS_PER_W - 1, 1)


@jax.jit
def kernel(batch_Phrase_emb, Phrase_type_ids, phrase_attribute_emb_all):
    ids = Phrase_type_ids.astype(jnp.int32)
    ids_pad = jnp.pad(ids, ((0, 0), (0, LP - L)), constant_values=1000)
    worker = (jnp.arange(B, dtype=jnp.int32) // ROWS_PER_W) % REP
    ids_off = (ids_pad + (worker * (NUM_TYPES + 1))[:, None]).reshape(-1)
    tab0 = jnp.tile(phrase_attribute_emb_all[:, :HH], (REP, 1))
    tab1 = jnp.tile(phrase_attribute_emb_all[:, HH:], (REP, 1))

    run = pl.kernel(
        _body,
        out_type=jax.ShapeDtypeStruct((B, L, H), jnp.float32),
        mesh=plsc.VectorSubcoreMesh(core_axis_name="c", subcore_axis_name="s"),
        scratch_types=[
            pltpu.VMEM((4, LP), jnp.int32),
            pltpu.VMEM((2, L, H), jnp.float32),
            pltpu.VMEM((2, L, HH), jnp.float32),
            pltpu.SemaphoreType.DMA((4,)),
            pltpu.SemaphoreType.DMA((2,)),
            pltpu.SemaphoreType.DMA((2,)),
            pltpu.SemaphoreType.DMA((2,)),
        ],
    )
    return run(batch_Phrase_emb, ids_off, tab0, tab1)
